# baseline (speedup 1.0000x reference)
import functools

import jax
import jax.numpy as jnp
from jax import lax
from jax.experimental import pallas as pl
from jax.experimental.pallas import tpu as pltpu
from jax.experimental.pallas import tpu_sc as plsc

M = 1000000
D = 32
DE = 16
B = 16384
FP = 128

NC = 2
NS = 16
BT = B // NS
IC = 128
NCHUNK = BT // IC

COLS_PT = 62464
MK = 999936
COLS_LAST = MK - (NS - 1) * COLS_PT
W = 1024
NCB = COLS_PT // W
LASTW = COLS_LAST - COLS_PT

TS_PT = 62504
TS_LAST = M - (NS - 1) * TS_PT
TSB = 8192

MCAP = B + 16


def _impl(memT, memory_ts, mailT, mailbox_ts, idx3, mail2p, ts2):
    mesh = plsc.VectorSubcoreMesh(core_axis_name="c", subcore_axis_name="s")

    @functools.partial(
        pl.kernel,
        mesh=mesh,
        out_type=(
            jax.ShapeDtypeStruct((D, M), jnp.float32),
            jax.ShapeDtypeStruct((M,), jnp.float32),
            jax.ShapeDtypeStruct((D + DE, M), jnp.float32),
            jax.ShapeDtypeStruct((M,), jnp.float32),
        ),
        scratch_types=[
            pltpu.VMEM((NS, NCHUNK, IC), jnp.int32),
            pltpu.VMEM((MCAP,), jnp.int32),
            pltpu.VMEM((4, 8, W), jnp.float32),
            pltpu.VMEM((16, FP), jnp.float32),
            pltpu.VMEM((NCHUNK, IC), jnp.int32),
            pltpu.VMEM((BT,), jnp.float32),
            pltpu.VMEM((2 * TSB,), jnp.float32),
            pltpu.SemaphoreType.DMA,
        ] + [pltpu.SemaphoreType.DMA] * 8,
        compiler_params=pltpu.CompilerParams(use_tc_tiling_on_sc=True,
                                            needs_layout_passes=False),
    )
    def k(memT_h, memts_h, mailT_h, mailts_h, idx3_h, mail2p_h, ts2_h,
          memT_o, memts_o, mailT_o, mailts_o,
          idx_all, mpos, ring, rows16, idx_v, ts_v, tsbuf,
          sem, *ring_sems):
        in_sems = ring_sems[:4]
        out_sems = ring_sems[4:]
        c = lax.axis_index("c")
        s = lax.axis_index("s")
        lanes = lax.iota(jnp.int32, 16)

        def ts_copy(src, dst, base, n, in_sem=None, out_sem=None):
            in_sem = in_sems[0] if in_sem is None else in_sem
            out_sem = out_sems[0] if out_sem is None else out_sem
            nfull = n // TSB
            outs = []
            for i in range(nfull):
                b = base + i * TSB
                half = tsbuf.at[pl.ds((i % 2) * TSB, TSB)]
                if i >= 2:
                    outs[i - 2].wait()
                pltpu.async_copy(src.at[pl.ds(b, TSB)], half, in_sem).wait()
                outs.append(pltpu.async_copy(
                    half, dst.at[pl.ds(b, TSB)], out_sem))
            rem = n - nfull * TSB
            if rem:
                b = base + nfull * TSB
                half = tsbuf.at[pl.ds((nfull % 2) * TSB, rem)]
                if nfull >= 2:
                    outs[nfull - 2].wait()
                pltpu.async_copy(src.at[pl.ds(b, rem)], half, in_sem).wait()
                outs.append(pltpu.async_copy(
                    half, dst.at[pl.ds(b, rem)], out_sem))
            for o in outs[max(0, len(outs) - 2):]:
                o.wait()

        t0 = s * TS_PT

        @pl.when(c == 0)
        def _():
            @pl.when(s < NS - 1)
            def _():
                ts_copy(memts_h, memts_o, t0, TS_PT)

            @pl.when(s == NS - 1)
            def _():
                ts_copy(memts_h, memts_o, t0, TS_LAST)

        @pl.when(c == 1)
        def _():
            @pl.when(s < NS - 1)
            def _():
                ts_copy(mailts_h, mailts_o, t0, TS_PT)

            @pl.when(s == NS - 1)
            def _():
                ts_copy(mailts_h, mailts_o, t0, TS_LAST)

        plsc.subcore_barrier()

        b0 = s * BT
        pltpu.sync_copy(idx3_h.at[s], idx_v)
        pltpu.sync_copy(ts2_h.at[pl.ds(b0, BT)], ts_v)

        @pl.when(c == 0)
        def _():
            for j in range(NCHUNK):
                pltpu.async_copy(ts_v.at[pl.ds(j * IC, IC)],
                                 memts_o.at[idx_v.at[j]], sem).wait()

        @pl.when(c == 1)
        def _():
            for j in range(NCHUNK):
                pltpu.async_copy(ts_v.at[pl.ds(j * IC, IC)],
                                 mailts_o.at[idx_v.at[j]], sem).wait()

        c0 = s * COLS_PT
        c1 = c0 + COLS_PT + (s == NS - 1).astype(jnp.int32) * (COLS_LAST - COLS_PT)
        pltpu.sync_copy(idx3_h, idx_all)

        def disc_body(g, off):
            v = idx_all[g >> 6, (g >> 3) & 7, pl.ds((g & 7) * 16, 16)]
            mk = (v >= c0) & (v < c1)
            p = lanes + g * 16
            mki = mk.astype(jnp.int32)
            rank = plsc.cumsum(mki) - mki
            dest = jnp.where(mk, off + rank, MCAP - 1)
            plsc.store_scatter(mpos, [dest], p)
            return off + jnp.sum(mki)

        m = lax.fori_loop(0, B // 16, disc_body, jnp.int32(0))
        mpos[pl.ds(m, 16)] = jnp.zeros((16,), jnp.int32)
        ngrp = (m + 15) // 16

        NB = 4

        def make_copy_patch(src, dst, F):
            ntr = F // 8
            nchk = ntr * NCB
            nsup = nchk // NB
            nrem = nchk - nsup * NB

            def patch(b, tr, cc0):
                tr8 = tr * 8

                def grp_body(g, _):
                    p = mpos[pl.ds(g * 16, 16)]
                    cols = plsc.load_gather(
                        idx_all, [p >> 10, (p >> 7) & 7, p & 127])
                    inchunk = (cols >= cc0) & (cols < cc0 + W)

                    @pl.when(jnp.any(inchunk))
                    def _():
                        ccm = cols - cc0
                        enc = jnp.min(jnp.where(inchunk, p * 1024 + ccm,
                                                jnp.int32(2**30)))
                        psel = jnp.where(inchunk, p, enc >> 10)
                        ccx = jnp.where(inchunk, ccm, enc & 1023)
                        pltpu.async_copy(mail2p_h.at[psel], rows16, sem).wait()
                        for fo in range(8):
                            fg = jnp.full((16,), tr8 + fo, jnp.int32)
                            fl = jnp.full((16,), fo, jnp.int32)
                            vals = plsc.load_gather(rows16, [lanes, fg])
                            plsc.store_scatter(ring.at[b], [fl, ccx], vals)
                    return 0

                lax.fori_loop(0, ngrp, grp_body, 0)

            def chunk_in(tr, cb, b):
                pltpu.async_copy(
                    src.at[pl.ds(tr * 8, 8), pl.ds(c0 + cb * W, W)],
                    ring.at[b], in_sems[b])

            def chunk_out(tr, cb, b):
                return pltpu.async_copy(
                    ring.at[b],
                    dst.at[pl.ds(tr * 8, 8), pl.ds(c0 + cb * W, W)],
                    out_sems[b])

            def chunk_drain(b, semref):
                pltpu.make_async_copy(
                    src.at[pl.ds(0, 8), pl.ds(c0, W)],
                    ring.at[b], semref).wait()

            def step(tr, cb):
                nx = cb + 1
                wrap = (nx == NCB).astype(jnp.int32)
                return tr + wrap, nx * (1 - wrap)

            def super_body(gs, carry):
                tr, cb, tr2, cb2 = carry
                for b in range(NB):
                    i = gs * NB + b
                    b2 = (b + 2) % NB

                    @pl.when(i >= 2)
                    def _():
                        chunk_drain(b2, out_sems[b2])

                    @pl.when(i + 2 < nchk)
                    def _():
                        chunk_in(tr2, cb2, b2)
                    chunk_drain(b, in_sems[b])
                    patch(b, tr, c0 + cb * W)
                    chunk_out(tr, cb, b)
                    tr, cb = step(tr, cb)
                    tr2, cb2 = step(tr2, cb2)
                return tr, cb, tr2, cb2

            chunk_in(0, 0, 0)
            chunk_in(0, 1, 1)
            z = jnp.int32(0)
            tr, cb, tr2, cb2 = lax.fori_loop(
                0, nsup, super_body, (z, z, z, z + 2))
            for r in range(nrem):
                i = nsup * NB + r
                b = i % NB
                chunk_drain(b, in_sems[b])
                patch(b, tr, c0 + cb * W)
                chunk_out(tr, cb, b).wait()
                tr, cb = step(tr, cb)
            chunk_drain((nsup * NB - 2) % NB, out_sems[(nsup * NB - 2) % NB])
            chunk_drain((nsup * NB - 1) % NB, out_sems[(nsup * NB - 1) % NB])

            def patch_last(tr, cc0):
                tr8 = tr * 8

                def grp_body(g, _):
                    p = mpos[pl.ds(g * 16, 16)]
                    cols = plsc.load_gather(
                        idx_all, [p >> 10, (p >> 7) & 7, p & 127])
                    inchunk = (cols >= cc0) & (cols < cc0 + LASTW)

                    @pl.when(jnp.any(inchunk))
                    def _():
                        ccm = cols - cc0
                        enc = jnp.min(jnp.where(inchunk, p * 1024 + ccm,
                                                jnp.int32(2**30)))
                        psel = jnp.where(inchunk, p, enc >> 10)
                        ccx = jnp.where(inchunk, ccm, enc & 1023)
                        pltpu.async_copy(mail2p_h.at[psel], rows16, sem).wait()
                        for fo in range(8):
                            fg = jnp.full((16,), tr8 + fo, jnp.int32)
                            fl = jnp.full((16,), fo, jnp.int32)
                            vals = plsc.load_gather(rows16, [lanes, fg])
                            plsc.store_scatter(ring.at[0], [fl, ccx], vals)
                    return 0

                lax.fori_loop(0, ngrp, grp_body, 0)

            @pl.when(s == NS - 1)
            def _():
                cc0 = c0 + NCB * W
                for trr in range(ntr):
                    pltpu.async_copy(
                        src.at[pl.ds(trr * 8, 8), pl.ds(cc0, LASTW)],
                        ring.at[0, :, pl.ds(0, LASTW)], in_sems[0]).wait()
                    patch_last(trr, cc0)
                    pltpu.async_copy(
                        ring.at[0, :, pl.ds(0, LASTW)],
                        dst.at[pl.ds(trr * 8, 8), pl.ds(cc0, LASTW)],
                        out_sems[0]).wait()

        @pl.when(c == 0)
        def _():
            make_copy_patch(memT_h, memT_o, D)

        @pl.when(c == 1)
        def _():
            make_copy_patch(mailT_h, mailT_o, D + DE)

    return k(memT, memory_ts, mailT, mailbox_ts, idx3, mail2p, ts2)


def kernel(memory, memory_ts, mailbox, mailbox_ts, idx, val, ts, edge_feats):
    iota = jnp.arange(B, dtype=jnp.int32)
    pos = jnp.full((M,), -1, dtype=jnp.int32).at[idx].max(iota)
    win = pos[idx]
    ts2 = ts[win]
    mail2p = jnp.concatenate(
        [val[win], edge_feats[win],
         jnp.zeros((B, FP - D - DE), jnp.float32)], axis=1)
    idx3 = idx.reshape(NS, NCHUNK, IC)
    mT, mts, bT, bts = _impl(memory.T, memory_ts, mailbox.T, mailbox_ts,
                             idx3, mail2p, ts2)
    new_memory, new_mailbox = mT.T, bT.T

    tidx = jnp.where(idx >= MK, idx - MK, M)
    tail_mem = memory[MK:].at[tidx].set(mail2p[:, :D], mode="drop")
    tail_mail = mailbox[MK:].at[tidx].set(mail2p[:, :D + DE], mode="drop")
    new_memory = new_memory.at[MK:].set(tail_mem)
    new_mailbox = new_mailbox.at[MK:].set(tail_mail)
    return new_memory, mts, new_mailbox, bts


# speedup vs baseline: 4.0891x; 4.0891x over previous
import functools

import jax
import jax.numpy as jnp
from jax import lax
from jax.experimental import pallas as pl
from jax.experimental.pallas import tpu as pltpu
from jax.experimental.pallas import tpu_sc as plsc

M = 1000000
D = 32
DE = 16
B = 16384

NC = 2
NS = 16
BT = B // NS
IC = 128
NCHUNK = BT // IC


def _scatter_inplace(mem_r, memts_r, mail_r, mailts_r, idx3, val2, mail2, ts2):
    mesh = plsc.VectorSubcoreMesh(core_axis_name="c", subcore_axis_name="s")

    @functools.partial(
        pl.kernel,
        mesh=mesh,
        scratch_types=[
            pltpu.VMEM((NCHUNK, IC), jnp.int32),
            pltpu.VMEM((BT, D + DE), jnp.float32),
            pltpu.VMEM((BT, D), jnp.float32),
            pltpu.VMEM((BT,), jnp.float32),
            pltpu.SemaphoreType.DMA,
        ],
        compiler_params=pltpu.CompilerParams(use_tc_tiling_on_sc=False),
    )
    def k(mem_o, memts_o, mail_o, mailts_o, idx3_h, val2_h, mail2_h, ts2_h,
          idx_v, mail_v, val_v, ts_v, sem):
        c = lax.axis_index("c")
        s = lax.axis_index("s")

        b0 = s * BT
        pltpu.sync_copy(idx3_h.at[s], idx_v)
        pltpu.sync_copy(ts2_h.at[pl.ds(b0, BT)], ts_v)

        @pl.when(c == 0)
        def _scat_mem():
            pltpu.sync_copy(val2_h.at[pl.ds(b0, BT)], val_v)
            for j in range(NCHUNK):
                pltpu.async_copy(
                    val_v.at[pl.ds(j * IC, IC)],
                    mem_o.at[idx_v.at[j]], sem).wait()
                pltpu.async_copy(
                    ts_v.at[pl.ds(j * IC, IC)],
                    memts_o.at[idx_v.at[j]], sem).wait()

        @pl.when(c == 1)
        def _scat_mail():
            pltpu.sync_copy(mail2_h.at[pl.ds(b0, BT)], mail_v)
            for j in range(NCHUNK):
                pltpu.async_copy(
                    mail_v.at[pl.ds(j * IC, IC)],
                    mail_o.at[idx_v.at[j]], sem).wait()
                pltpu.async_copy(
                    ts_v.at[pl.ds(j * IC, IC)],
                    mailts_o.at[idx_v.at[j]], sem).wait()

    return k(mem_r, memts_r, mail_r, mailts_r, idx3, val2, mail2, ts2)


def kernel(memory, memory_ts, mailbox, mailbox_ts, idx, val, ts, edge_feats):
    iota = jnp.arange(B, dtype=jnp.int32)
    pos = jnp.full((M,), -1, dtype=jnp.int32).at[idx].max(iota)
    win = pos[idx]
    val2 = val[win]
    mail2 = jnp.concatenate([val2, edge_feats[win]], axis=1)
    ts2 = ts[win]
    idx3 = idx.reshape(NS, NCHUNK, IC)

    mem_r = jax.new_ref(memory)
    memts_r = jax.new_ref(memory_ts)
    mail_r = jax.new_ref(mailbox)
    mailts_r = jax.new_ref(mailbox_ts)
    _scatter_inplace(mem_r, memts_r, mail_r, mailts_r, idx3, val2, mail2, ts2)
    return mem_r[...], memts_r[...], mail_r[...], mailts_r[...]
